# Initial kernel scaffold; baseline (speedup 1.0000x reference)
import jax
import jax.numpy as jnp
from jax.experimental import pallas as pl


def kernel(x, gate_w, expert_bias, w_gate, w_value, w_out):
    raise NotImplementedError("write your pallas kernel here")


# speedup vs baseline: 6.7032x; 6.7032x over previous
import functools

import jax
import jax.numpy as jnp
from jax import lax
from jax.experimental import pallas as pl
from jax.experimental.pallas import tpu as pltpu
from jax.experimental.pallas import tpu_sc as plsc

D_MODEL = 768
NUM_EXPERTS = 64
EXPERT_DIM = 256
CAP = 128
N_TOKENS = 2048
TBLK = 256
NB = N_TOKENS // TBLK
S_SLOTS = NUM_EXPERTS * CAP

NC = 2
NS = 16
L = 16
NW = NC * NS
SLOTS_PER_W = S_SLOTS // NW
GCHUNK = 128
TOK_PER_W = N_TOKENS // NW


def _router_body(x_ref, gw_ref, bias_ref, pos_ref, wgt_ref, cnt_ref):
    b = pl.program_id(0)

    @pl.when(b == 0)
    def _():
        cnt_ref[...] = jnp.zeros_like(cnt_ref)

    xb = x_ref[...]
    gw = gw_ref[...]
    logits = lax.dot_general(xb, gw, (((1,), (1,)), ((), ())),
                             preferred_element_type=jnp.float32)
    logits = logits + bias_ref[...]
    m = jnp.max(logits, axis=1, keepdims=True)
    s = jnp.sum(jnp.exp(logits - m), axis=1)
    p = 1.0 / s
    w = p / (p + 1e-8)

    col = lax.broadcasted_iota(jnp.int32, (TBLK, NUM_EXPERTS), 1)
    sel = jnp.min(jnp.where(logits == m, col, NUM_EXPERTS), axis=1)
    onehot = (col == sel[:, None]).astype(jnp.float32)

    c = onehot
    d = 1
    while d < TBLK:
        shifted = jnp.concatenate(
            [jnp.zeros((d, NUM_EXPERTS), jnp.float32), c[: TBLK - d]], axis=0)
        c = c + shifted
        d *= 2

    run = cnt_ref[...]
    rank = jnp.sum(onehot * (c + run), axis=1) - 1.0
    cnt_ref[...] = run + jnp.sum(onehot, axis=0, keepdims=True)

    rank_i = rank.astype(jnp.int32)
    dropped = rank_i >= CAP
    pos = sel * CAP + jnp.where(dropped, 0, rank_i)
    wgt = jnp.where(dropped, 0.0, w)
    pos_ref[...] = pos.reshape(1, 1, TBLK)
    wgt_ref[...] = wgt.reshape(1, 1, TBLK)


def _router(xf, gate_w, expert_bias):
    pos3, wgt3 = pl.pallas_call(
        _router_body,
        grid=(NB,),
        in_specs=[
            pl.BlockSpec((TBLK, D_MODEL), lambda b: (b, 0)),
            pl.BlockSpec((NUM_EXPERTS, D_MODEL), lambda b: (0, 0)),
            pl.BlockSpec((1, NUM_EXPERTS), lambda b: (0, 0)),
        ],
        out_specs=[
            pl.BlockSpec((1, 1, TBLK), lambda b: (b, 0, 0)),
            pl.BlockSpec((1, 1, TBLK), lambda b: (b, 0, 0)),
        ],
        out_shape=[
            jax.ShapeDtypeStruct((NB, 1, TBLK), jnp.int32),
            jax.ShapeDtypeStruct((NB, 1, TBLK), jnp.float32),
        ],
        scratch_shapes=[pltpu.VMEM((1, NUM_EXPERTS), jnp.float32)],
    )(xf, gate_w, expert_bias.reshape(1, NUM_EXPERTS))
    return pos3.reshape(N_TOKENS), wgt3.reshape(N_TOKENS)


def _dispatch_body(pos_hbm, x_hbm, xs_hbm, pos_v, tok_v, rows_v, sem):
    wid = lax.axis_index("s") * NC + lax.axis_index("c")
    base = wid * SLOTS_PER_W
    pltpu.sync_copy(pos_hbm, pos_v)

    for j in range(SLOTS_PER_W // L):
        tok_v[pl.ds(j * L, L)] = jnp.zeros((L,), jnp.int32)

    lanes = lax.broadcasted_iota(jnp.int32, (L,), 0)

    def scatter_step(i, carry):
        p = pos_v[pl.ds(i * L, L)]
        rel = p - base
        msk = (rel >= 0) & (rel < SLOTS_PER_W)
        relc = jnp.where(msk, rel, 0)
        toks = lanes + i * L
        plsc.store_scatter(tok_v, [relc], toks, mask=msk)
        return carry

    lax.fori_loop(0, N_TOKENS // L, scatter_step, 0)

    for c in range(SLOTS_PER_W // GCHUNK):
        pltpu.async_copy(x_hbm.at[tok_v.at[pl.ds(c * GCHUNK, GCHUNK)]],
                         rows_v, sem).wait()
        pltpu.sync_copy(rows_v, xs_hbm.at[pl.ds(base + c * GCHUNK, GCHUNK)])


def _dispatch(pos, xf):
    mesh = plsc.VectorSubcoreMesh(core_axis_name="c", subcore_axis_name="s")
    f = functools.partial(
        pl.kernel,
        mesh=mesh,
        out_type=jax.ShapeDtypeStruct((S_SLOTS, D_MODEL), jnp.float32),
        compiler_params=pltpu.CompilerParams(needs_layout_passes=False),
        scratch_types=[
            pltpu.VMEM((N_TOKENS,), jnp.int32),
            pltpu.VMEM((SLOTS_PER_W,), jnp.int32),
            pltpu.VMEM((GCHUNK, D_MODEL), jnp.float32),
            pltpu.SemaphoreType.DMA,
        ],
    )(_dispatch_body)
    return f(pos, xf)


def _experts_body(xs_ref, wg_ref, wv_ref, wo_ref, h_ref):
    xe = xs_ref[...]
    wg = wg_ref[0]
    wv = wv_ref[0]
    wo = wo_ref[0]
    g = lax.dot_general(xe, wg, (((1,), (1,)), ((), ())),
                        preferred_element_type=jnp.float32)
    v = lax.dot_general(xe, wv, (((1,), (1,)), ((), ())),
                        preferred_element_type=jnp.float32)
    u = (g / (1.0 + jnp.exp(-g))) * v
    h_ref[...] = lax.dot_general(u, wo, (((1,), (1,)), ((), ())),
                                 preferred_element_type=jnp.float32)


def _experts(xs, w_gate, w_value, w_out):
    return pl.pallas_call(
        _experts_body,
        grid=(NUM_EXPERTS,),
        in_specs=[
            pl.BlockSpec((CAP, D_MODEL), lambda e: (e, 0)),
            pl.BlockSpec((1, EXPERT_DIM, D_MODEL), lambda e: (e, 0, 0)),
            pl.BlockSpec((1, EXPERT_DIM, D_MODEL), lambda e: (e, 0, 0)),
            pl.BlockSpec((1, D_MODEL, EXPERT_DIM), lambda e: (e, 0, 0)),
        ],
        out_specs=pl.BlockSpec((CAP, D_MODEL), lambda e: (e, 0)),
        out_shape=jax.ShapeDtypeStruct((S_SLOTS, D_MODEL), jnp.float32),
        compiler_params=pltpu.CompilerParams(
            dimension_semantics=("arbitrary",)),
    )(xs, w_gate, w_value, w_out)


def _combine_body(pos_hbm, wgt_hbm, h_hbm, out_hbm, pos_v, wgt_v, rows_v, sem):
    wid = lax.axis_index("s") * NC + lax.axis_index("c")
    tb = wid * TOK_PER_W
    pltpu.sync_copy(pos_hbm.at[pl.ds(tb, TOK_PER_W)], pos_v)
    pltpu.sync_copy(wgt_hbm.at[pl.ds(tb, TOK_PER_W)], wgt_v)
    pltpu.async_copy(h_hbm.at[pos_v], rows_v, sem).wait()

    def scale_row(i, carry):
        wv = plsc.load_gather(wgt_v, [jnp.broadcast_to(i, (L,))])
        for j in range(D_MODEL // L):
            sl = pl.ds(j * L, L)
            rows_v[i, sl] = rows_v[i, sl] * wv
        return carry

    lax.fori_loop(0, TOK_PER_W, scale_row, 0)
    pltpu.sync_copy(rows_v, out_hbm.at[pl.ds(tb, TOK_PER_W)])


def _combine(pos, wgt, h):
    mesh = plsc.VectorSubcoreMesh(core_axis_name="c", subcore_axis_name="s")
    f = functools.partial(
        pl.kernel,
        mesh=mesh,
        out_type=jax.ShapeDtypeStruct((N_TOKENS, D_MODEL), jnp.float32),
        compiler_params=pltpu.CompilerParams(needs_layout_passes=False),
        scratch_types=[
            pltpu.VMEM((TOK_PER_W,), jnp.int32),
            pltpu.VMEM((TOK_PER_W,), jnp.float32),
            pltpu.VMEM((TOK_PER_W, D_MODEL), jnp.float32),
            pltpu.SemaphoreType.DMA,
        ],
    )(_combine_body)
    return f(pos, wgt, h)


def kernel(x, gate_w, expert_bias, w_gate, w_value, w_out):
    B_, T_, D_ = x.shape
    xf = x.reshape(T_ * B_, D_)
    pos, wgt = _router(xf, gate_w, expert_bias)
    xs = _dispatch(pos, xf)
    h = _experts(xs, w_gate, w_value, w_out)
    out = _combine(pos, wgt, h)
    return out.reshape(B_, T_, D_)


# speedup vs baseline: 18.4238x; 2.7485x over previous
import functools

import jax
import jax.numpy as jnp
from jax import lax
from jax.experimental import pallas as pl
from jax.experimental.pallas import tpu as pltpu
from jax.experimental.pallas import tpu_sc as plsc

D_MODEL = 768
NUM_EXPERTS = 64
EXPERT_DIM = 256
CAP = 128
N_TOKENS = 2048
TBLK = 256
NB = N_TOKENS // TBLK
S_SLOTS = NUM_EXPERTS * CAP

NC = 2
NS = 16
L = 16
NW = NC * NS
SLOTS_PER_W = S_SLOTS // NW
GCHUNK = 128
TOK_PER_W = N_TOKENS // NW


def _router_body(x_ref, gw_ref, bias_ref, pos_ref, wgt_ref, cnt_ref):
    b = pl.program_id(0)

    @pl.when(b == 0)
    def _():
        cnt_ref[...] = jnp.zeros_like(cnt_ref)

    xb = x_ref[...]
    gw = gw_ref[...]
    logits = lax.dot_general(xb, gw, (((1,), (1,)), ((), ())),
                             preferred_element_type=jnp.float32)
    logits = logits + bias_ref[...]
    m = jnp.max(logits, axis=1, keepdims=True)
    s = jnp.sum(jnp.exp(logits - m), axis=1)
    p = 1.0 / s
    w = p / (p + 1e-8)

    col = lax.broadcasted_iota(jnp.int32, (TBLK, NUM_EXPERTS), 1)
    sel = jnp.min(jnp.where(logits == m, col, NUM_EXPERTS), axis=1)
    onehot = (col == sel[:, None]).astype(jnp.float32)

    c = onehot
    d = 1
    while d < TBLK:
        shifted = jnp.concatenate(
            [jnp.zeros((d, NUM_EXPERTS), jnp.float32), c[: TBLK - d]], axis=0)
        c = c + shifted
        d *= 2

    run = cnt_ref[...]
    rank = jnp.sum(onehot * (c + run), axis=1) - 1.0
    cnt_ref[...] = run + jnp.sum(onehot, axis=0, keepdims=True)

    rank_i = rank.astype(jnp.int32)
    dropped = rank_i >= CAP
    pos = sel * CAP + jnp.where(dropped, 0, rank_i)
    wgt = jnp.where(dropped, 0.0, w)
    pos_ref[...] = pos.reshape(1, 1, TBLK)
    wgt_ref[...] = wgt.reshape(1, 1, TBLK)


def _router(xf, gate_w, expert_bias):
    pos3, wgt3 = pl.pallas_call(
        _router_body,
        grid=(NB,),
        in_specs=[
            pl.BlockSpec((TBLK, D_MODEL), lambda b: (b, 0)),
            pl.BlockSpec((NUM_EXPERTS, D_MODEL), lambda b: (0, 0)),
            pl.BlockSpec((1, NUM_EXPERTS), lambda b: (0, 0)),
        ],
        out_specs=[
            pl.BlockSpec((1, 1, TBLK), lambda b: (b, 0, 0)),
            pl.BlockSpec((1, 1, TBLK), lambda b: (b, 0, 0)),
        ],
        out_shape=[
            jax.ShapeDtypeStruct((NB, 1, TBLK), jnp.int32),
            jax.ShapeDtypeStruct((NB, 1, TBLK), jnp.float32),
        ],
        scratch_shapes=[pltpu.VMEM((1, NUM_EXPERTS), jnp.float32)],
    )(xf, gate_w, expert_bias.reshape(1, NUM_EXPERTS))
    return pos3.reshape(N_TOKENS), wgt3.reshape(N_TOKENS)


def _dispatch_body(pos_hbm, x_hbm, xs_hbm, pos_v, tok0_v, tok1_v, rows_v, sem):
    wid = lax.axis_index("s") * NC + lax.axis_index("c")
    base = wid * SLOTS_PER_W
    pltpu.sync_copy(pos_hbm, pos_v)

    lanes = lax.broadcasted_iota(jnp.int32, (L,), 0)

    for j in range(GCHUNK // L):
        fill = (base + j * L + lanes) & (N_TOKENS - 1)
        tok0_v[pl.ds(j * L, L)] = fill
        tok1_v[pl.ds(j * L, L)] = (fill + GCHUNK) & (N_TOKENS - 1)

    def scatter_step(i, carry):
        p = pos_v[pl.ds(i * L, L)]
        rel = p - base
        m0 = (rel >= 0) & (rel < GCHUNK)
        m1 = (rel >= GCHUNK) & (rel < SLOTS_PER_W)
        toks = lanes + i * L
        plsc.store_scatter(tok0_v, [jnp.where(m0, rel, 0)], toks, mask=m0)
        plsc.store_scatter(tok1_v, [jnp.where(m1, rel - GCHUNK, 0)], toks,
                           mask=m1)
        return carry

    lax.fori_loop(0, N_TOKENS // L, scatter_step, 0)

    for c, tok_v in enumerate((tok0_v, tok1_v)):
        pltpu.async_copy(x_hbm.at[tok_v], rows_v, sem).wait()
        pltpu.sync_copy(rows_v, xs_hbm.at[pl.ds(base + c * GCHUNK, GCHUNK)])


def _dispatch(pos, xf):
    mesh = plsc.VectorSubcoreMesh(core_axis_name="c", subcore_axis_name="s")
    f = functools.partial(
        pl.kernel,
        mesh=mesh,
        out_type=jax.ShapeDtypeStruct((S_SLOTS, D_MODEL), jnp.float32),
        compiler_params=pltpu.CompilerParams(needs_layout_passes=False),
        scratch_types=[
            pltpu.VMEM((N_TOKENS,), jnp.int32),
            pltpu.VMEM((GCHUNK,), jnp.int32),
            pltpu.VMEM((GCHUNK,), jnp.int32),
            pltpu.VMEM((GCHUNK, D_MODEL), jnp.float32),
            pltpu.SemaphoreType.DMA,
        ],
    )(_dispatch_body)
    return f(pos, xf)


def _experts_body(xs_ref, wg_ref, wv_ref, wo_ref, h_ref):
    xe = xs_ref[...]
    wg = wg_ref[0]
    wv = wv_ref[0]
    wo = wo_ref[0]
    g = lax.dot_general(xe, wg, (((1,), (1,)), ((), ())),
                        preferred_element_type=jnp.float32)
    v = lax.dot_general(xe, wv, (((1,), (1,)), ((), ())),
                        preferred_element_type=jnp.float32)
    u = (g / (1.0 + jnp.exp(-g))) * v
    h_ref[...] = lax.dot_general(u, wo, (((1,), (1,)), ((), ())),
                                 preferred_element_type=jnp.float32)


def _experts(xs, w_gate, w_value, w_out):
    return pl.pallas_call(
        _experts_body,
        grid=(NUM_EXPERTS,),
        in_specs=[
            pl.BlockSpec((CAP, D_MODEL), lambda e: (e, 0)),
            pl.BlockSpec((1, EXPERT_DIM, D_MODEL), lambda e: (e, 0, 0)),
            pl.BlockSpec((1, EXPERT_DIM, D_MODEL), lambda e: (e, 0, 0)),
            pl.BlockSpec((1, D_MODEL, EXPERT_DIM), lambda e: (e, 0, 0)),
        ],
        out_specs=pl.BlockSpec((CAP, D_MODEL), lambda e: (e, 0)),
        out_shape=jax.ShapeDtypeStruct((S_SLOTS, D_MODEL), jnp.float32),
        compiler_params=pltpu.CompilerParams(
            dimension_semantics=("arbitrary",)),
    )(xs, w_gate, w_value, w_out)


def _combine_body(pos_hbm, wgt_hbm, h_hbm, out_hbm, pos_v, wgt_v, rows_v, sem):
    wid = lax.axis_index("s") * NC + lax.axis_index("c")
    tb = wid * TOK_PER_W
    pltpu.sync_copy(pos_hbm.at[pl.ds(tb, TOK_PER_W)], pos_v)
    pltpu.sync_copy(wgt_hbm.at[pl.ds(tb, TOK_PER_W)], wgt_v)
    pltpu.async_copy(h_hbm.at[pos_v], rows_v, sem).wait()

    def scale_row(i, carry):
        wv = plsc.load_gather(wgt_v, [jnp.broadcast_to(i, (L,))])
        for j in range(D_MODEL // L):
            sl = pl.ds(j * L, L)
            rows_v[i, sl] = rows_v[i, sl] * wv
        return carry

    lax.fori_loop(0, TOK_PER_W, scale_row, 0)
    pltpu.sync_copy(rows_v, out_hbm.at[pl.ds(tb, TOK_PER_W)])


def _combine(pos, wgt, h):
    mesh = plsc.VectorSubcoreMesh(core_axis_name="c", subcore_axis_name="s")
    f = functools.partial(
        pl.kernel,
        mesh=mesh,
        out_type=jax.ShapeDtypeStruct((N_TOKENS, D_MODEL), jnp.float32),
        compiler_params=pltpu.CompilerParams(needs_layout_passes=False),
        scratch_types=[
            pltpu.VMEM((TOK_PER_W,), jnp.int32),
            pltpu.VMEM((TOK_PER_W,), jnp.float32),
            pltpu.VMEM((TOK_PER_W, D_MODEL), jnp.float32),
            pltpu.SemaphoreType.DMA,
        ],
    )(_combine_body)
    return f(pos, wgt, h)


def kernel(x, gate_w, expert_bias, w_gate, w_value, w_out):
    B_, T_, D_ = x.shape
    xf = x.reshape(T_ * B_, D_)
    pos, wgt = _router(xf, gate_w, expert_bias)
    xs = _dispatch(pos, xf)
    h = _experts(xs, w_gate, w_value, w_out)
    out = _combine(pos, wgt, h)
    return out.reshape(B_, T_, D_)
